# baseline (speedup 1.0000x reference)
import jax
import jax.numpy as jnp
import numpy as np
from jax import lax
from jax.experimental import pallas as pl
from jax.experimental.pallas import tpu as pltpu
from jax.experimental.pallas import tpu_sc as plsc

_B = 64
_EMBED = 768
_NVIS = 256
_NMASK = 768
_NROWS = 1025
_NW = 32
_RB = 41
_NBLK = _NROWS // _RB
_NPAD = 1040


def _build_maps(sidx_ref, midx_ref, src_ref):
    def init(i, _):
        src_ref[i] = -1
        return 0

    lax.fori_loop(0, _NPAD, init, 0)
    src_ref[0] = 0

    def samp(j, _):
        src_ref[sidx_ref[j]] = j + 1
        return 0

    lax.fori_loop(0, _NVIS, samp, 0)

    def msk(j, _):
        src_ref[midx_ref[j]] = -2
        return 0

    lax.fori_loop(0, _NMASK, msk, 0)


def _sc_body(x_hbm, src_hbm, w0_hbm, b_hbm, out_hbm,
             srcb, stg0, stg1, wbuf, bbuf, semg, semo):
    cid = lax.axis_index("c")
    sid = lax.axis_index("s")
    wid = sid * 2 + cid
    pltpu.sync_copy(src_hbm, srcb)
    pltpu.sync_copy(w0_hbm, wbuf)
    pltpu.sync_copy(b_hbm.at[pl.ds(0, 16)], bbuf)
    acc = jnp.zeros((16,), jnp.float32)
    for k in range(_EMBED // 16):
        acc = acc + wbuf[pl.ds(k * 16, 16)]
    wsum = acc[0]
    for k in range(1, 16):
        wsum = wsum + acc[k]
    b0 = bbuf[...][0]
    s_val = wsum * np.float32(127.0 / 255.0) + b0
    sv = jnp.full((16,), s_val, jnp.float32)
    zv = jnp.zeros((16,), jnp.float32)

    def block(k, stg, oc):
        b = 2 * wid + k // _NBLK
        t = lax.rem(k, _NBLK)
        r0 = t * _RB

        @pl.when(oc >= 2)
        def _():
            pltpu.make_async_copy(
                stg, out_hbm.at[pl.ds(0, _RB)], semo
            ).wait()

        def row(i, gcnt):
            src_i = srcb[pl.ds(r0 + i, 16)][0]
            full = jnp.logical_and(src_i >= 0, gcnt >= 16)

            @pl.when(full)
            def _():
                pltpu.make_async_copy(
                    x_hbm.at[pl.ds(0, 1)], stg.at[pl.ds(0, 1)], semg
                ).wait()

            @pl.when(src_i >= 0)
            def _():
                pltpu.async_copy(
                    x_hbm.at[pl.ds(b * (1 + _NVIS) + src_i, 1)],
                    stg.at[pl.ds(i, 1)],
                    semg,
                )

            @pl.when(src_i == -1)
            def _():
                for c in range(_EMBED // 16):
                    stg[i, pl.ds(c * 16, 16)] = zv

            @pl.when(src_i == -2)
            def _():
                for c in range(_EMBED // 16):
                    stg[i, pl.ds(c * 16, 16)] = sv

            return gcnt - full.astype(jnp.int32) + (src_i >= 0).astype(jnp.int32)

        gcnt = lax.fori_loop(0, _RB, row, 0)

        def draing(j, c):
            pltpu.make_async_copy(
                x_hbm.at[pl.ds(0, 1)], stg.at[pl.ds(0, 1)], semg
            ).wait()
            return c

        lax.fori_loop(0, gcnt, draing, 0)

        pltpu.async_copy(
            stg, out_hbm.at[pl.ds(b * _NROWS + r0, _RB)], semo
        )

    def blk_body(k, oc):
        @pl.when(lax.rem(k, 2) == 0)
        def _():
            block(k, stg0, oc)

        @pl.when(lax.rem(k, 2) == 1)
        def _():
            block(k, stg1, oc)

        return jnp.minimum(oc + 1, 2)

    oc = lax.fori_loop(0, 2 * _NBLK, blk_body, 0)

    def draino(j, c):
        pltpu.make_async_copy(stg0, out_hbm.at[pl.ds(0, _RB)], semo).wait()
        return c

    lax.fori_loop(0, jnp.minimum(oc, 2), draino, 0)


def kernel(x, sample_index, mask_index, W, b):
    src = pl.pallas_call(
        _build_maps,
        in_specs=[
            pl.BlockSpec(memory_space=pltpu.SMEM),
            pl.BlockSpec(memory_space=pltpu.SMEM),
        ],
        out_specs=pl.BlockSpec(memory_space=pltpu.SMEM),
        out_shape=jax.ShapeDtypeStruct((_NPAD,), jnp.int32),
    )(sample_index, mask_index)

    x2d = jnp.reshape(x, (_B * (1 + _NVIS), _EMBED))
    w0 = jnp.reshape(W[0], (_EMBED,))

    mesh = plsc.VectorSubcoreMesh(core_axis_name="c", subcore_axis_name="s")
    out2 = pl.kernel(
        _sc_body,
        out_type=jax.ShapeDtypeStruct((_B * _NROWS, _EMBED), jnp.float32),
        mesh=mesh,
        compiler_params=pltpu.CompilerParams(use_tc_tiling_on_sc=False),
        scratch_types=[
            pltpu.VMEM((_NPAD,), jnp.int32),
            pltpu.VMEM((_RB, _EMBED), jnp.float32),
            pltpu.VMEM((_RB, _EMBED), jnp.float32),
            pltpu.VMEM((_EMBED,), jnp.float32),
            pltpu.VMEM((16,), jnp.float32),
            pltpu.SemaphoreType.DMA,
            pltpu.SemaphoreType.DMA,
        ],
    )(x2d, src, w0, b)

    return jnp.reshape(out2, (_B, _NROWS, _EMBED))
